# baseline (speedup 1.0000x reference)
import functools

import jax
import jax.numpy as jnp
from jax import lax
from jax.experimental import pallas as pl
from jax.experimental.pallas import tpu as pltpu
from jax.experimental.pallas import tpu_sc as plsc

NC = 2
NS = 16
NW = NC * NS
D = 128
SC_K = 128
NBUF = 2


def _make_sc_agg(n_pad: int, epw: int, with_cnt: bool):
    K = SC_K
    n_chunks = epw // K
    rpt = n_pad // NS

    mesh = plsc.VectorSubcoreMesh(core_axis_name="c", subcore_axis_name="s")

    out_type = [jax.ShapeDtypeStruct((NC, n_pad, D), jnp.float32)]
    scratch = [
        [pltpu.VMEM((2, K), jnp.int32) for _ in range(NBUF)],
        [pltpu.VMEM((K, D), jnp.float32) for _ in range(NBUF)],
        pltpu.VMEM_SHARED((n_pad, D), jnp.float32),
        pltpu.SemaphoreType.DMA,
        pltpu.SemaphoreType.DMA,
    ]
    if with_cnt:
        out_type.append(jax.ShapeDtypeStruct((NW, 1, n_pad), jnp.float32))
        scratch.insert(2, pltpu.VMEM((1, n_pad), jnp.float32))

    @functools.partial(
        pl.kernel,
        mesh=mesh,
        out_type=tuple(out_type),
        scratch_types=scratch,
        compiler_params=pltpu.CompilerParams(needs_layout_passes=False),
    )
    def agg(x_hbm, ei_hbm, z_hbm, zc_hbm, *rest):
        if with_cnt:
            p_hbm, cnt_hbm, idx, rows, hist, acc, gsem, isem = rest
        else:
            p_hbm, idx, rows, acc, gsem, isem = rest
            cnt_hbm = hist = None
        c = lax.axis_index("c")
        s = lax.axis_index("s")
        wid = c * NS + s

        pltpu.sync_copy(z_hbm, acc.at[pl.ds(s * rpt, rpt)])
        if with_cnt:
            pltpu.sync_copy(zc_hbm, hist)
        plsc.subcore_barrier()

        zero16 = jnp.zeros((16,), jnp.int32)
        ones16 = jnp.ones((16,), jnp.float32)
        lane = lax.iota(jnp.int32, 16)

        def idx_copy(i, b):
            r0 = (wid * n_chunks + i) * 2
            return pltpu.make_async_copy(ei_hbm.at[pl.ds(r0, 2)], idx[b], isem)

        def gather(i, b):
            return pltpu.make_async_copy(x_hbm.at[idx[b].at[0]], rows[b], gsem)

        def step(i, b, first=False, last=False, fire=True):
            nb = (b + 1) % NBUF
            gather(i, b).wait()
            pltpu.sync_copy(rows[b], acc.at[idx[b].at[1]], add=True)
            if not last:
                if not first:
                    idx_copy(i + 1, nb).wait()
                gather(i + 1, nb).start()
            if with_cnt:
                for j in range(K // 16):
                    d = idx[b][1, pl.ds(j * 16, 16)]
                    for l in range(16):
                        plsc.addupdate_scatter(
                            hist, [zero16, d], ones16, mask=lane == l)
            if fire and not last:
                idx_copy(i + 2, b).start()

        pltpu.sync_copy(ei_hbm.at[pl.ds(wid * n_chunks * 2, 2)], idx[0])
        pltpu.sync_copy(ei_hbm.at[pl.ds((wid * n_chunks + 1) * 2, 2)], idx[1])
        gather(0, 0).start()
        step(0, 0, first=True)

        def body(m, carry):
            for u in range(NBUF):
                step(2 * m + 1 + u, (1 + u) % NBUF)
            return carry

        lax.fori_loop(0, (n_chunks - 4) // 2, body, 0)
        step(n_chunks - 3, (n_chunks - 3) % NBUF)
        step(n_chunks - 2, (n_chunks - 2) % NBUF, fire=False)
        step(n_chunks - 1, (n_chunks - 1) % NBUF, last=True)

        if with_cnt:
            pltpu.sync_copy(hist, cnt_hbm.at[wid])
        plsc.subcore_barrier()
        pltpu.sync_copy(acc.at[pl.ds(s * rpt, rpt)],
                        p_hbm.at[c, pl.ds(s * rpt, rpt)])

    return agg


def _make_tc_layer(n_pad: int, relu: bool):
    blk = 2048
    grid = (n_pad // blk,)

    def body(p_ref, cnt_ref, x_ref, wl_ref, b_ref, wr_ref, o_ref):
        ssum = p_ref[0] + p_ref[1]
        cnt = jnp.sum(cnt_ref[...], axis=(0, 1))
        cc = jnp.reshape(cnt, (blk, 1))
        mean = ssum / jnp.maximum(cc, 1.0)
        y = (jnp.dot(mean, wl_ref[...], preferred_element_type=jnp.float32)
             + b_ref[...]
             + jnp.dot(x_ref[...], wr_ref[...],
                       preferred_element_type=jnp.float32))
        if relu:
            y = jnp.maximum(y, 0.0)
        o_ref[...] = y

    return pl.pallas_call(
        body,
        grid=grid,
        in_specs=[
            pl.BlockSpec((NC, blk, D), lambda i: (0, i, 0)),
            pl.BlockSpec((NW, 1, blk), lambda i: (0, 0, i)),
            pl.BlockSpec((blk, D), lambda i: (i, 0)),
            pl.BlockSpec((D, D), lambda i: (0, 0)),
            pl.BlockSpec((1, D), lambda i: (0, 0)),
            pl.BlockSpec((D, D), lambda i: (0, 0)),
        ],
        out_specs=pl.BlockSpec((blk, D), lambda i: (i, 0)),
        out_shape=jax.ShapeDtypeStruct((n_pad, D), jnp.float32),
    )


def kernel(x, edge_index, W1l, b1, W1r, W2l, b2, W2r):
    n_nodes, d = x.shape
    n_edges = edge_index.shape[1]
    n_pad = -(-n_nodes // (128 * NS)) * (128 * NS)
    epw = -(-(n_edges // NW) // (SC_K * NBUF)) * (SC_K * NBUF)
    e_pad = epw * NW
    n_chunks = epw // SC_K
    src = jnp.zeros((e_pad,), jnp.int32).at[:n_edges].set(
        edge_index[0].astype(jnp.int32))
    dst = jnp.full((e_pad,), n_nodes, jnp.int32).at[:n_edges].set(
        edge_index[1].astype(jnp.int32))
    ei = jnp.stack([src.reshape(NW, n_chunks, SC_K),
                    dst.reshape(NW, n_chunks, SC_K)],
                   axis=2).reshape(NW * n_chunks * 2, SC_K)

    zrows = jnp.zeros((n_pad // NS, D), jnp.float32)
    zcnt = jnp.zeros((1, n_pad), jnp.float32)

    p1, cnt = _make_sc_agg(n_pad, epw, with_cnt=True)(
        x, ei, zrows, zcnt)
    h = _make_tc_layer(n_pad, relu=True)(
        p1, cnt, jnp.zeros((n_pad, D), jnp.float32).at[:n_nodes].set(x),
        W1l, b1.reshape(1, D), W1r)
    (p2,) = _make_sc_agg(n_pad, epw, with_cnt=False)(
        h, ei, zrows, zcnt)
    out = _make_tc_layer(n_pad, relu=False)(
        p2, cnt, h, W2l, b2.reshape(1, D), W2r)
    return out[:n_nodes]


# speedup vs baseline: 1.0002x; 1.0002x over previous
import functools

import jax
import jax.numpy as jnp
from jax import lax
from jax.experimental import pallas as pl
from jax.experimental.pallas import tpu as pltpu
from jax.experimental.pallas import tpu_sc as plsc

NC = 2
NS = 16
NW = NC * NS
D = 128
SC_K = 128
NBUF = 2


def _make_sc_agg(n_pad: int, epw: int, with_cnt: bool):
    K = SC_K
    n_chunks = epw // K
    rpt = n_pad // NS

    mesh = plsc.VectorSubcoreMesh(core_axis_name="c", subcore_axis_name="s")

    out_type = [jax.ShapeDtypeStruct((NC, n_pad, D), jnp.float32)]
    scratch = [
        [pltpu.VMEM((2, K), jnp.int32) for _ in range(NBUF)],
        [pltpu.VMEM((K, D), jnp.float32) for _ in range(NBUF)],
        pltpu.VMEM_SHARED((n_pad, D), jnp.float32),
        pltpu.SemaphoreType.DMA,
        pltpu.SemaphoreType.DMA,
    ]
    if with_cnt:
        out_type.append(jax.ShapeDtypeStruct((NW, 1, n_pad), jnp.float32))
        scratch.insert(2, pltpu.VMEM((1, n_pad), jnp.float32))

    @functools.partial(
        pl.kernel,
        mesh=mesh,
        out_type=tuple(out_type),
        scratch_types=scratch,
        compiler_params=pltpu.CompilerParams(needs_layout_passes=False),
    )
    def agg(x_hbm, ei_hbm, z_hbm, zc_hbm, *rest):
        if with_cnt:
            p_hbm, cnt_hbm, idx, rows, hist, acc, gsem, isem = rest
        else:
            p_hbm, idx, rows, acc, gsem, isem = rest
            cnt_hbm = hist = None
        c = lax.axis_index("c")
        s = lax.axis_index("s")
        wid = c * NS + s

        pltpu.sync_copy(z_hbm, acc.at[pl.ds(s * rpt, rpt)])
        if with_cnt:
            pltpu.sync_copy(zc_hbm, hist)
        plsc.subcore_barrier()

        zero16 = jnp.zeros((16,), jnp.int32)
        ones16 = jnp.ones((16,), jnp.float32)
        lane = lax.iota(jnp.int32, 16)

        def idx_copy(i, b):
            r0 = (wid * n_chunks + i) * 2
            return pltpu.make_async_copy(ei_hbm.at[pl.ds(r0, 2)], idx[b], isem)

        def gather(i, b):
            return pltpu.make_async_copy(x_hbm.at[idx[b].at[0]], rows[b], gsem)

        def step(i, b, first=False, last=False, fire=True):
            nb = (b + 1) % NBUF
            gather(i, b).wait()
            pltpu.sync_copy(rows[b], acc.at[idx[b].at[1]], add=True)
            if not last:
                if not first:
                    idx_copy(i + 1, nb).wait()
                gather(i + 1, nb).start()
            if with_cnt:
                for j in range(K // 16):
                    d = idx[b][1, pl.ds(j * 16, 16)]
                    for l in range(16):
                        plsc.addupdate_scatter(
                            hist, [zero16, d], ones16, mask=lane == l)
            if fire and not last:
                idx_copy(i + 2, b).start()

        pltpu.sync_copy(ei_hbm.at[pl.ds(wid * n_chunks * 2, 2)], idx[0])
        pltpu.sync_copy(ei_hbm.at[pl.ds((wid * n_chunks + 1) * 2, 2)], idx[1])
        gather(0, 0).start()
        step(0, 0, first=True)

        def body(m, carry):
            for u in range(NBUF):
                step(2 * m + 1 + u, (1 + u) % NBUF)
            return carry

        lax.fori_loop(0, (n_chunks - 4) // 2, body, 0)
        step(n_chunks - 3, (n_chunks - 3) % NBUF)
        step(n_chunks - 2, (n_chunks - 2) % NBUF, fire=False)
        step(n_chunks - 1, (n_chunks - 1) % NBUF, last=True)

        if with_cnt:
            pltpu.sync_copy(hist, cnt_hbm.at[wid])
        plsc.subcore_barrier()
        pltpu.sync_copy(acc.at[pl.ds(s * rpt, rpt)],
                        p_hbm.at[c, pl.ds(s * rpt, rpt)])

    return agg


def _make_tc_layer(n_pad: int, relu: bool):
    blk = 2048
    grid = (n_pad // blk,)

    def body(p_ref, cnt_ref, x_ref, wl_ref, b_ref, wr_ref, o_ref):
        ssum = p_ref[0] + p_ref[1]
        cnt = jnp.sum(cnt_ref[...], axis=(0, 1))
        cc = jnp.reshape(cnt, (blk, 1))
        mean = ssum / jnp.maximum(cc, 1.0)
        y = (jnp.dot(mean, wl_ref[...], preferred_element_type=jnp.float32)
             + b_ref[...]
             + jnp.dot(x_ref[...], wr_ref[...],
                       preferred_element_type=jnp.float32))
        if relu:
            y = jnp.maximum(y, 0.0)
        o_ref[...] = y

    return pl.pallas_call(
        body,
        grid=grid,
        in_specs=[
            pl.BlockSpec((NC, blk, D), lambda i: (0, i, 0)),
            pl.BlockSpec((NW, 1, blk), lambda i: (0, 0, i)),
            pl.BlockSpec((blk, D), lambda i: (i, 0)),
            pl.BlockSpec((D, D), lambda i: (0, 0)),
            pl.BlockSpec((1, D), lambda i: (0, 0)),
            pl.BlockSpec((D, D), lambda i: (0, 0)),
        ],
        out_specs=pl.BlockSpec((blk, D), lambda i: (i, 0)),
        out_shape=jax.ShapeDtypeStruct((n_pad, D), jnp.float32),
    )


def kernel(x, edge_index, W1l, b1, W1r, W2l, b2, W2r):
    n_nodes, d = x.shape
    n_edges = edge_index.shape[1]
    n_pad = -(-n_nodes // (128 * NS)) * (128 * NS)
    epw = -(-(n_edges // NW) // (SC_K * NBUF)) * (SC_K * NBUF)
    e_pad = epw * NW
    n_chunks = epw // SC_K
    src = jnp.zeros((e_pad,), jnp.int32).at[:n_edges].set(
        edge_index[0].astype(jnp.int32))
    pad_dst = n_nodes + jnp.arange(e_pad - n_edges, dtype=jnp.int32) % (
        n_pad - n_nodes)
    dst = jnp.concatenate(
        [edge_index[1].astype(jnp.int32), pad_dst])
    ei = jnp.stack([src.reshape(NW, n_chunks, SC_K),
                    dst.reshape(NW, n_chunks, SC_K)],
                   axis=2).reshape(NW * n_chunks * 2, SC_K)

    zrows = jnp.zeros((n_pad // NS, D), jnp.float32)
    zcnt = jnp.zeros((1, n_pad), jnp.float32)

    p1, cnt = _make_sc_agg(n_pad, epw, with_cnt=True)(
        x, ei, zrows, zcnt)
    h = _make_tc_layer(n_pad, relu=True)(
        p1, cnt, jnp.zeros((n_pad, D), jnp.float32).at[:n_nodes].set(x),
        W1l, b1.reshape(1, D), W1r)
    (p2,) = _make_sc_agg(n_pad, epw, with_cnt=False)(
        h, ei, zrows, zcnt)
    out = _make_tc_layer(n_pad, relu=False)(
        p2, cnt, h, W2l, b2.reshape(1, D), W2r)
    return out[:n_nodes]


# speedup vs baseline: 1.6492x; 1.6489x over previous
import functools

import jax
import jax.numpy as jnp
from jax import lax
from jax.experimental import pallas as pl
from jax.experimental.pallas import tpu as pltpu
from jax.experimental.pallas import tpu_sc as plsc

NC = 2
NS = 16
NW = NC * NS
D = 128
SC_K = 80
NBUF = 2


def _make_sc_agg(n_pad: int, epw: int, with_cnt: bool):
    K = SC_K
    n_chunks = epw // K
    rpt = n_pad // NS

    mesh = plsc.VectorSubcoreMesh(core_axis_name="c", subcore_axis_name="s")

    out_type = [jax.ShapeDtypeStruct((NC, n_pad, D), jnp.float32)]
    scratch = [
        [pltpu.VMEM((2, K), jnp.int32) for _ in range(NBUF)],
        [pltpu.VMEM((K, D), jnp.float32) for _ in range(NBUF)],
        pltpu.VMEM_SHARED((n_pad, D), jnp.float32),
        pltpu.SemaphoreType.DMA,
        pltpu.SemaphoreType.DMA,
    ]
    if with_cnt:
        out_type.append(jax.ShapeDtypeStruct((NW, 1, n_pad), jnp.float32))
        scratch.insert(2, pltpu.VMEM((1, n_pad), jnp.float32))

    @functools.partial(
        pl.kernel,
        mesh=mesh,
        out_type=tuple(out_type),
        scratch_types=scratch,
        compiler_params=pltpu.CompilerParams(needs_layout_passes=False),
    )
    def agg(x_hbm, ei_hbm, z_hbm, zc_hbm, *rest):
        if with_cnt:
            p_hbm, cnt_hbm, idx, rows, hist, acc, gsem, isem = rest
        else:
            p_hbm, idx, rows, acc, gsem, isem = rest
            cnt_hbm = hist = None
        c = lax.axis_index("c")
        s = lax.axis_index("s")
        wid = c * NS + s

        pltpu.sync_copy(z_hbm, acc.at[pl.ds(s * rpt, rpt)])
        if with_cnt:
            pltpu.sync_copy(zc_hbm, hist)
        plsc.subcore_barrier()

        zero16 = jnp.zeros((16,), jnp.int32)
        ones16 = jnp.ones((16,), jnp.float32)
        lane = lax.iota(jnp.int32, 16)

        def idx_copy(i, b):
            r0 = (wid * n_chunks + i) * 2
            return pltpu.make_async_copy(ei_hbm.at[pl.ds(r0, 2)], idx[b], isem)

        def gather(i, b):
            return pltpu.make_async_copy(x_hbm.at[idx[b].at[0]], rows[b], gsem)

        def step(i, b, first=False, last=False, fire=True):
            nb = (b + 1) % NBUF
            gather(i, b).wait()
            pltpu.sync_copy(rows[b], acc.at[idx[b].at[1]], add=True)
            if not last:
                if not first:
                    idx_copy(i + 1, nb).wait()
                gather(i + 1, nb).start()
            if with_cnt:
                for j in range(K // 16):
                    d = idx[b][1, pl.ds(j * 16, 16)]
                    for l in range(16):
                        plsc.addupdate_scatter(
                            hist, [zero16, d], ones16, mask=lane == l)
            if fire and not last:
                idx_copy(i + 2, b).start()

        pltpu.sync_copy(ei_hbm.at[pl.ds(wid * n_chunks * 2, 2)], idx[0])
        pltpu.sync_copy(ei_hbm.at[pl.ds((wid * n_chunks + 1) * 2, 2)], idx[1])
        gather(0, 0).start()
        step(0, 0, first=True)

        def body(m, carry):
            for u in range(NBUF):
                step(2 * m + 1 + u, (1 + u) % NBUF)
            return carry

        lax.fori_loop(0, (n_chunks - 4) // 2, body, 0)
        step(n_chunks - 3, (n_chunks - 3) % NBUF)
        step(n_chunks - 2, (n_chunks - 2) % NBUF, fire=False)
        step(n_chunks - 1, (n_chunks - 1) % NBUF, last=True)

        if with_cnt:
            pltpu.sync_copy(hist, cnt_hbm.at[wid])
        plsc.subcore_barrier()
        pltpu.sync_copy(acc.at[pl.ds(s * rpt, rpt)],
                        p_hbm.at[c, pl.ds(s * rpt, rpt)])

    return agg


def _make_tc_layer(n_pad: int, relu: bool):
    blk = 2048
    grid = (n_pad // blk,)

    def body(p_ref, cnt_ref, x_ref, wl_ref, b_ref, wr_ref, o_ref):
        ssum = p_ref[0] + p_ref[1]
        cnt = jnp.sum(cnt_ref[...], axis=(0, 1))
        cc = jnp.reshape(cnt, (blk, 1))
        mean = ssum / jnp.maximum(cc, 1.0)
        y = (jnp.dot(mean, wl_ref[...], preferred_element_type=jnp.float32)
             + b_ref[...]
             + jnp.dot(x_ref[...], wr_ref[...],
                       preferred_element_type=jnp.float32))
        if relu:
            y = jnp.maximum(y, 0.0)
        o_ref[...] = y

    return pl.pallas_call(
        body,
        grid=grid,
        in_specs=[
            pl.BlockSpec((NC, blk, D), lambda i: (0, i, 0)),
            pl.BlockSpec((NW, 1, blk), lambda i: (0, 0, i)),
            pl.BlockSpec((blk, D), lambda i: (i, 0)),
            pl.BlockSpec((D, D), lambda i: (0, 0)),
            pl.BlockSpec((1, D), lambda i: (0, 0)),
            pl.BlockSpec((D, D), lambda i: (0, 0)),
        ],
        out_specs=pl.BlockSpec((blk, D), lambda i: (i, 0)),
        out_shape=jax.ShapeDtypeStruct((n_pad, D), jnp.float32),
    )


def kernel(x, edge_index, W1l, b1, W1r, W2l, b2, W2r):
    n_nodes, d = x.shape
    n_edges = edge_index.shape[1]
    n_pad = -(-n_nodes // (128 * NS)) * (128 * NS)
    epw = -(-(n_edges // NW) // (SC_K * NBUF)) * (SC_K * NBUF)
    e_pad = epw * NW
    n_chunks = epw // SC_K
    src = jnp.zeros((e_pad,), jnp.int32).at[:n_edges].set(
        edge_index[0].astype(jnp.int32))
    pad_dst = n_nodes + jnp.arange(e_pad - n_edges, dtype=jnp.int32) % (
        n_pad - n_nodes)
    dst = jnp.concatenate(
        [edge_index[1].astype(jnp.int32), pad_dst])
    ei = jnp.stack([src.reshape(NW, n_chunks, SC_K),
                    dst.reshape(NW, n_chunks, SC_K)],
                   axis=2).reshape(NW * n_chunks * 2, SC_K)

    zrows = jnp.zeros((n_pad // NS, D), jnp.float32)
    zcnt = jnp.zeros((1, n_pad), jnp.float32)

    p1, cnt = _make_sc_agg(n_pad, epw, with_cnt=True)(
        x, ei, zrows, zcnt)
    h = _make_tc_layer(n_pad, relu=True)(
        p1, cnt, jnp.zeros((n_pad, D), jnp.float32).at[:n_nodes].set(x),
        W1l, b1.reshape(1, D), W1r)
    (p2,) = _make_sc_agg(n_pad, epw, with_cnt=False)(
        h, ei, zrows, zcnt)
    out = _make_tc_layer(n_pad, relu=False)(
        p2, cnt, h, W2l, b2.reshape(1, D), W2r)
    return out[:n_nodes]


# speedup vs baseline: 1.7412x; 1.0558x over previous
import functools

import jax
import jax.numpy as jnp
from jax import lax
from jax.experimental import pallas as pl
from jax.experimental.pallas import tpu as pltpu
from jax.experimental.pallas import tpu_sc as plsc

NC = 2
NS = 16
NW = NC * NS
D = 128


def _make_sc_agg(n_pad: int, n_edges: int, with_cnt: bool):
    epw = n_edges // NW
    K = 80
    n_chunks = epw // K
    rpt = n_pad // NS

    mesh = plsc.VectorSubcoreMesh(core_axis_name="c", subcore_axis_name="s")

    out_type = [jax.ShapeDtypeStruct((NC, n_pad, D), jnp.float32)]
    scratch = [
        pltpu.VMEM((K,), jnp.int32),
        pltpu.VMEM((K,), jnp.int32),
        pltpu.VMEM((K, D), jnp.float32),
        pltpu.VMEM_SHARED((n_pad, D), jnp.float32),
        pltpu.SemaphoreType.DMA,
    ]
    if with_cnt:
        out_type.append(jax.ShapeDtypeStruct((NW, 1, n_pad), jnp.float32))
        scratch.insert(3, pltpu.VMEM((1, n_pad), jnp.float32))

    @functools.partial(
        pl.kernel,
        mesh=mesh,
        out_type=tuple(out_type),
        scratch_types=scratch,
        compiler_params=pltpu.CompilerParams(needs_layout_passes=False),
    )
    def agg(x_hbm, src_hbm, dst_hbm, z_hbm, zc_hbm, *rest):
        if with_cnt:
            p_hbm, cnt_hbm, isrc, idst, rows, hist, acc, sem = rest
        else:
            p_hbm, isrc, idst, rows, acc, sem = rest
            cnt_hbm = hist = None
        c = lax.axis_index("c")
        s = lax.axis_index("s")
        wid = c * NS + s

        pltpu.sync_copy(z_hbm, acc.at[pl.ds(s * rpt, rpt)])
        if with_cnt:
            pltpu.sync_copy(zc_hbm, hist)
        plsc.subcore_barrier()

        zero16 = jnp.zeros((16,), jnp.int32)
        ones16 = jnp.ones((16,), jnp.float32)
        lane = lax.iota(jnp.int32, 16)

        def body(i, carry):
            eb = wid * epw + i * K
            pltpu.sync_copy(src_hbm.at[pl.ds(eb, K)], isrc)
            pltpu.sync_copy(dst_hbm.at[pl.ds(eb, K)], idst)
            pltpu.async_copy(x_hbm.at[isrc], rows, sem).wait()
            pltpu.sync_copy(rows, acc.at[idst], add=True)
            if with_cnt:
                for j in range(K // 16):
                    d = idst[pl.ds(j * 16, 16)]
                    for l in range(16):
                        plsc.addupdate_scatter(
                            hist, [zero16, d], ones16, mask=lane == l)
            return carry

        lax.fori_loop(0, n_chunks, body, 0)
        if with_cnt:
            pltpu.sync_copy(hist, cnt_hbm.at[wid])
        plsc.subcore_barrier()
        pltpu.sync_copy(acc.at[pl.ds(s * rpt, rpt)],
                        p_hbm.at[c, pl.ds(s * rpt, rpt)])

    return agg


def _make_tc_layer(n_pad: int, relu: bool):
    blk = 2048
    grid = (n_pad // blk,)

    def body(p_ref, cnt_ref, x_ref, wl_ref, b_ref, wr_ref, o_ref):
        ssum = p_ref[0] + p_ref[1]
        cnt = jnp.sum(cnt_ref[...], axis=(0, 1))
        cc = jnp.reshape(cnt, (blk, 1))
        mean = ssum / jnp.maximum(cc, 1.0)
        y = (jnp.dot(mean, wl_ref[...], preferred_element_type=jnp.float32)
             + b_ref[...]
             + jnp.dot(x_ref[...], wr_ref[...],
                       preferred_element_type=jnp.float32))
        if relu:
            y = jnp.maximum(y, 0.0)
        o_ref[...] = y

    return pl.pallas_call(
        body,
        grid=grid,
        in_specs=[
            pl.BlockSpec((NC, blk, D), lambda i: (0, i, 0)),
            pl.BlockSpec((NW, 1, blk), lambda i: (0, 0, i)),
            pl.BlockSpec((blk, D), lambda i: (i, 0)),
            pl.BlockSpec((D, D), lambda i: (0, 0)),
            pl.BlockSpec((1, D), lambda i: (0, 0)),
            pl.BlockSpec((D, D), lambda i: (0, 0)),
        ],
        out_specs=pl.BlockSpec((blk, D), lambda i: (i, 0)),
        out_shape=jax.ShapeDtypeStruct((n_pad, D), jnp.float32),
    )


def kernel(x, edge_index, W1l, b1, W1r, W2l, b2, W2r):
    n_nodes, d = x.shape
    n_edges = edge_index.shape[1]
    n_pad = -(-n_nodes // (128 * NS)) * (128 * NS)
    src = edge_index[0].astype(jnp.int32)
    dst = edge_index[1].astype(jnp.int32)

    zrows = jnp.zeros((n_pad // NS, D), jnp.float32)
    zcnt = jnp.zeros((1, n_pad), jnp.float32)

    p1, cnt = _make_sc_agg(n_pad, n_edges, with_cnt=True)(
        x, src, dst, zrows, zcnt)
    h = _make_tc_layer(n_pad, relu=True)(
        p1, cnt, jnp.zeros((n_pad, D), jnp.float32).at[:n_nodes].set(x),
        W1l, b1.reshape(1, D), W1r)
    (p2,) = _make_sc_agg(n_pad, n_edges, with_cnt=False)(
        h, src, dst, zrows, zcnt)
    out = _make_tc_layer(n_pad, relu=False)(
        p2, cnt, h, W2l, b2.reshape(1, D), W2r)
    return out[:n_nodes]
